# Initial kernel scaffold; baseline (speedup 1.0000x reference)
import jax
import jax.numpy as jnp
from jax.experimental import pallas as pl


def kernel(fields, sides, species, moves, items, abilities, move_attributes, pokemon_attributes, species_table, item_table, ability_table, move_table):
    raise NotImplementedError("write your pallas kernel here")


# speedup vs baseline: 2.2041x; 2.2041x over previous
import jax
import jax.numpy as jnp
from jax import lax
from jax.experimental import pallas as pl
from jax.experimental.pallas import tpu as pltpu
from jax.experimental.pallas import tpu_sc as plsc

_B = 4096
_R = _B * 2 * 6
_NW = 32
_PER_W = _R // _NW
_CH = 128
_NCH = _PER_W // _CH

_SPECIES_DIM = 128
_ITEM_DIM = 64
_ABILITY_DIM = 64
_MOVE_DIM = 128
_MA_DIM = 32
_PA_DIM = 48
_ROW = 848

_OFF_SP = 0
_OFF_IT = 128
_OFF_AB = 192
_OFF_MV = 256
_OFF_MA = 768
_OFF_PA = 800


def _body(sp_idx_h, it_idx_h, ab_idx_h, mv_idx_h, ma_h, pa_h,
          sp_t, it_t, ab_t, mv_t, out_h,
          idx_sp, idx_it, idx_ab, idx_m0, idx_m1, idx_m2, idx_m3,
          r_sp, r_it, r_ab, r_m0, r_m1, r_m2, r_m3, r_ma, r_pa, sem):
    wid = lax.axis_index("s") * 2 + lax.axis_index("c")

    def step(i, carry):
        base = wid * _PER_W + i * _CH
        rows = pl.ds(base, _CH)
        loads = [
            pltpu.async_copy(sp_idx_h.at[rows], idx_sp, sem),
            pltpu.async_copy(it_idx_h.at[rows], idx_it, sem),
            pltpu.async_copy(ab_idx_h.at[rows], idx_ab, sem),
            pltpu.async_copy(mv_idx_h.at[0, rows], idx_m0, sem),
            pltpu.async_copy(mv_idx_h.at[1, rows], idx_m1, sem),
            pltpu.async_copy(mv_idx_h.at[2, rows], idx_m2, sem),
            pltpu.async_copy(mv_idx_h.at[3, rows], idx_m3, sem),
            pltpu.async_copy(ma_h.at[rows], r_ma, sem),
            pltpu.async_copy(pa_h.at[rows], r_pa, sem),
        ]
        for c in loads:
            c.wait()
        gathers = [
            pltpu.async_copy(sp_t.at[idx_sp], r_sp, sem),
            pltpu.async_copy(it_t.at[idx_it], r_it, sem),
            pltpu.async_copy(ab_t.at[idx_ab], r_ab, sem),
            pltpu.async_copy(mv_t.at[idx_m0], r_m0, sem),
            pltpu.async_copy(mv_t.at[idx_m1], r_m1, sem),
            pltpu.async_copy(mv_t.at[idx_m2], r_m2, sem),
            pltpu.async_copy(mv_t.at[idx_m3], r_m3, sem),
        ]
        for c in gathers:
            c.wait()
        stores = [
            pltpu.async_copy(r_sp, out_h.at[rows, pl.ds(_OFF_SP, _SPECIES_DIM)], sem),
            pltpu.async_copy(r_it, out_h.at[rows, pl.ds(_OFF_IT, _ITEM_DIM)], sem),
            pltpu.async_copy(r_ab, out_h.at[rows, pl.ds(_OFF_AB, _ABILITY_DIM)], sem),
            pltpu.async_copy(r_m0, out_h.at[rows, pl.ds(_OFF_MV + 0 * _MOVE_DIM, _MOVE_DIM)], sem),
            pltpu.async_copy(r_m1, out_h.at[rows, pl.ds(_OFF_MV + 1 * _MOVE_DIM, _MOVE_DIM)], sem),
            pltpu.async_copy(r_m2, out_h.at[rows, pl.ds(_OFF_MV + 2 * _MOVE_DIM, _MOVE_DIM)], sem),
            pltpu.async_copy(r_m3, out_h.at[rows, pl.ds(_OFF_MV + 3 * _MOVE_DIM, _MOVE_DIM)], sem),
            pltpu.async_copy(r_ma, out_h.at[rows, pl.ds(_OFF_MA, _MA_DIM)], sem),
            pltpu.async_copy(r_pa, out_h.at[rows, pl.ds(_OFF_PA, _PA_DIM)], sem),
        ]
        for c in stores:
            c.wait()
        return carry

    lax.fori_loop(0, _NCH, step, 0)


@jax.jit
def _encode(sp_idx, it_idx, ab_idx, mv_idx, ma, pa, sp_t, it_t, ab_t, mv_t):
    mesh = plsc.VectorSubcoreMesh(core_axis_name="c", subcore_axis_name="s")
    return pl.kernel(
        _body,
        out_type=jax.ShapeDtypeStruct((_R, _ROW), jnp.float32),
        mesh=mesh,
        compiler_params=pltpu.CompilerParams(use_tc_tiling_on_sc=False),
        scratch_types=[
            pltpu.VMEM((_CH,), jnp.int32),
            pltpu.VMEM((_CH,), jnp.int32),
            pltpu.VMEM((_CH,), jnp.int32),
            pltpu.VMEM((_CH,), jnp.int32),
            pltpu.VMEM((_CH,), jnp.int32),
            pltpu.VMEM((_CH,), jnp.int32),
            pltpu.VMEM((_CH,), jnp.int32),
            pltpu.VMEM((_CH, _SPECIES_DIM), jnp.float32),
            pltpu.VMEM((_CH, _ITEM_DIM), jnp.float32),
            pltpu.VMEM((_CH, _ABILITY_DIM), jnp.float32),
            pltpu.VMEM((_CH, _MOVE_DIM), jnp.float32),
            pltpu.VMEM((_CH, _MOVE_DIM), jnp.float32),
            pltpu.VMEM((_CH, _MOVE_DIM), jnp.float32),
            pltpu.VMEM((_CH, _MOVE_DIM), jnp.float32),
            pltpu.VMEM((_CH, _MA_DIM), jnp.float32),
            pltpu.VMEM((_CH, _PA_DIM), jnp.float32),
            pltpu.SemaphoreType.DMA,
        ],
    )(sp_idx, it_idx, ab_idx, mv_idx, ma, pa, sp_t, it_t, ab_t, mv_t)


def kernel(fields, sides, species, moves, items, abilities, move_attributes,
           pokemon_attributes, species_table, item_table, ability_table,
           move_table):
    sp_idx = species.reshape(_R).astype(jnp.int32)
    it_idx = items.reshape(_R).astype(jnp.int32)
    ab_idx = abilities.reshape(_R).astype(jnp.int32)
    mv_idx = moves.reshape(_R, 4).astype(jnp.int32).T
    ma = move_attributes.reshape(_R, _MA_DIM)
    pa = pokemon_attributes.reshape(_R, _PA_DIM)
    out = _encode(sp_idx, it_idx, ab_idx, mv_idx, ma, pa,
                  species_table, item_table, ability_table, move_table)
    return (fields, sides, out.reshape(_B, 2, 6, _ROW))


# speedup vs baseline: 2.2367x; 1.0148x over previous
import jax
import jax.numpy as jnp
from jax import lax
from jax.experimental import pallas as pl
from jax.experimental.pallas import tpu as pltpu
from jax.experimental.pallas import tpu_sc as plsc

_B = 4096
_R = _B * 2 * 6
_NW = 32
_PER_W = _R // _NW
_CH = 64
_NCH = _PER_W // _CH

_SPECIES_DIM = 128
_ITEM_DIM = 64
_ABILITY_DIM = 64
_MOVE_DIM = 128
_MA_DIM = 32
_PA_DIM = 48
_ROW = 848

_OFF_SP = 0
_OFF_IT = 128
_OFF_AB = 192
_OFF_MV = 256
_OFF_MA = 768
_OFF_PA = 800

_DIMS = (_SPECIES_DIM, _ITEM_DIM, _ABILITY_DIM,
         _MOVE_DIM, _MOVE_DIM, _MOVE_DIM, _MOVE_DIM)
_OFFS = (_OFF_SP, _OFF_IT, _OFF_AB,
         _OFF_MV, _OFF_MV + _MOVE_DIM, _OFF_MV + 2 * _MOVE_DIM,
         _OFF_MV + 3 * _MOVE_DIM)


def _body(pk_idx_h, ma_h, pa_h, sp_t, it_t, ab_t, mv_t, out_h, *refs):
    bufs = []
    for b in range(2):
        (idx_v, r_ma, r_pa, r_sp, r_it, r_ab, r_m0, r_m1, r_m2, r_m3,
         sem_l, sem_g, sem_s, sem_a) = refs[b * 14:(b + 1) * 14]
        bufs.append(dict(
            idx=idx_v, ma=r_ma, pa=r_pa,
            rows=(r_sp, r_it, r_ab, r_m0, r_m1, r_m2, r_m3),
            sl=sem_l, sg=sem_g, ss=sem_s, sa=sem_a))
    tabs = (sp_t, it_t, ab_t, mv_t, mv_t, mv_t, mv_t)
    wid = lax.axis_index("s") * 2 + lax.axis_index("c")
    w0 = wid * _PER_W

    def issue_l(i, s):
        rows = pl.ds(w0 + i * _CH, _CH)
        pltpu.async_copy(pk_idx_h.at[:, rows], s["idx"], s["sl"])
        pltpu.async_copy(ma_h.at[rows], s["ma"], s["sl"])
        pltpu.async_copy(pa_h.at[rows], s["pa"], s["sl"])

    def wait_l(s):
        rows = pl.ds(0, _CH)
        pltpu.make_async_copy(pk_idx_h.at[:, rows], s["idx"], s["sl"]).wait()
        pltpu.make_async_copy(ma_h.at[rows], s["ma"], s["sl"]).wait()
        pltpu.make_async_copy(pa_h.at[rows], s["pa"], s["sl"]).wait()

    def issue_g(s):
        for j in range(7):
            pltpu.async_copy(tabs[j].at[s["idx"].at[j]], s["rows"][j], s["sg"])

    def wait_g(s):
        for j in range(7):
            pltpu.make_async_copy(tabs[j].at[s["idx"].at[j]], s["rows"][j],
                                  s["sg"]).wait()

    def issue_s(i, s):
        rows = pl.ds(w0 + i * _CH, _CH)
        for j in range(7):
            pltpu.async_copy(s["rows"][j], out_h.at[rows, pl.ds(_OFFS[j], _DIMS[j])],
                             s["ss"])
        pltpu.async_copy(s["ma"], out_h.at[rows, pl.ds(_OFF_MA, _MA_DIM)], s["sa"])
        pltpu.async_copy(s["pa"], out_h.at[rows, pl.ds(_OFF_PA, _PA_DIM)], s["sa"])

    def wait_s(s):
        rows = pl.ds(0, _CH)
        for j in range(7):
            pltpu.make_async_copy(s["rows"][j],
                                  out_h.at[rows, pl.ds(_OFFS[j], _DIMS[j])],
                                  s["ss"]).wait()

    def wait_a(s):
        rows = pl.ds(0, _CH)
        pltpu.make_async_copy(s["ma"], out_h.at[rows, pl.ds(_OFF_MA, _MA_DIM)],
                              s["sa"]).wait()
        pltpu.make_async_copy(s["pa"], out_h.at[rows, pl.ds(_OFF_PA, _PA_DIM)],
                              s["sa"]).wait()

    issue_l(0, bufs[0])

    def step(g, carry):
        for b in range(2):
            i = 2 * g + b
            s, o = bufs[b], bufs[1 - b]
            wait_l(s)

            @pl.when(i >= 2)
            def _():
                wait_s(s)
            issue_g(s)

            @pl.when(i + 1 < _NCH)
            def _():
                @pl.when(i >= 1)
                def _():
                    wait_a(o)
                issue_l(i + 1, o)
            wait_g(s)
            issue_s(i, s)
        return carry

    lax.fori_loop(0, _NCH // 2, step, 0)
    wait_s(bufs[0])
    wait_s(bufs[1])
    wait_a(bufs[0])
    wait_a(bufs[1])


@jax.jit
def _encode(pk_idx, ma, pa, sp_t, it_t, ab_t, mv_t):
    mesh = plsc.VectorSubcoreMesh(core_axis_name="c", subcore_axis_name="s")
    per_buf = [
        pltpu.VMEM((7, _CH), jnp.int32),
        pltpu.VMEM((_CH, _MA_DIM), jnp.float32),
        pltpu.VMEM((_CH, _PA_DIM), jnp.float32),
        pltpu.VMEM((_CH, _SPECIES_DIM), jnp.float32),
        pltpu.VMEM((_CH, _ITEM_DIM), jnp.float32),
        pltpu.VMEM((_CH, _ABILITY_DIM), jnp.float32),
        pltpu.VMEM((_CH, _MOVE_DIM), jnp.float32),
        pltpu.VMEM((_CH, _MOVE_DIM), jnp.float32),
        pltpu.VMEM((_CH, _MOVE_DIM), jnp.float32),
        pltpu.VMEM((_CH, _MOVE_DIM), jnp.float32),
        pltpu.SemaphoreType.DMA,
        pltpu.SemaphoreType.DMA,
        pltpu.SemaphoreType.DMA,
        pltpu.SemaphoreType.DMA,
    ]
    return pl.kernel(
        _body,
        out_type=jax.ShapeDtypeStruct((_R, _ROW), jnp.float32),
        mesh=mesh,
        compiler_params=pltpu.CompilerParams(use_tc_tiling_on_sc=False),
        scratch_types=per_buf + per_buf,
    )(pk_idx, ma, pa, sp_t, it_t, ab_t, mv_t)


def kernel(fields, sides, species, moves, items, abilities, move_attributes,
           pokemon_attributes, species_table, item_table, ability_table,
           move_table):
    sp_idx = species.reshape(1, _R).astype(jnp.int32)
    it_idx = items.reshape(1, _R).astype(jnp.int32)
    ab_idx = abilities.reshape(1, _R).astype(jnp.int32)
    mv_idx = moves.reshape(_R, 4).astype(jnp.int32).T
    pk_idx = jnp.concatenate([sp_idx, it_idx, ab_idx, mv_idx], axis=0)
    ma = move_attributes.reshape(_R, _MA_DIM)
    pa = pokemon_attributes.reshape(_R, _PA_DIM)
    out = _encode(pk_idx, ma, pa,
                  species_table, item_table, ability_table, move_table)
    return (fields, sides, out.reshape(_B, 2, 6, _ROW))


# speedup vs baseline: 3.2462x; 1.4514x over previous
import jax
import jax.numpy as jnp
from jax import lax
from jax.experimental import pallas as pl
from jax.experimental.pallas import tpu as pltpu
from jax.experimental.pallas import tpu_sc as plsc

_B = 4096
_NW = 32
_NB = _B // _NW
_NBT = _B // 128
_NSLOT = 12
_ROW = 848
_NCT = _ROW // 8

_GW = 896
_GC_SP = 0
_GC_MV = 128
_GC_IT = 640
_GC_AB = 768

_MA_DIM = 32
_PA_DIM = 48


def _gather_body(idx_h, sp_t, mv_t, it_t, ab_t, g_h,
                 idx_v, bsp, bm0, bm1, bm2, bm3, bit, bab, sem):
    wid = lax.axis_index("s") * 2 + lax.axis_index("c")
    b0 = wid * _NB

    pltpu.sync_copy(idx_h.at[:, pl.ds(b0, _NB)], idx_v)

    for slot in range(_NSLOT):
        s, p = slot // 6, slot % 6
        q_sp = p * 2 + s
        gathers = [
            pltpu.async_copy(sp_t.at[idx_v.at[q_sp]], bsp, sem),
            pltpu.async_copy(it_t.at[idx_v.at[12 + q_sp]], bit, sem),
            pltpu.async_copy(ab_t.at[idx_v.at[24 + q_sp]], bab, sem),
            pltpu.async_copy(mv_t.at[idx_v.at[36 + slot * 4 + 0]], bm0, sem),
            pltpu.async_copy(mv_t.at[idx_v.at[36 + slot * 4 + 1]], bm1, sem),
            pltpu.async_copy(mv_t.at[idx_v.at[36 + slot * 4 + 2]], bm2, sem),
            pltpu.async_copy(mv_t.at[idx_v.at[36 + slot * 4 + 3]], bm3, sem),
        ]
        for c in gathers:
            c.wait()
        rows = pl.ds(slot * _B + b0, _NB)
        stores = [
            pltpu.async_copy(bsp, g_h.at[rows, pl.ds(_GC_SP, 128)], sem),
            pltpu.async_copy(bm0, g_h.at[rows, pl.ds(_GC_MV + 0 * 128, 128)], sem),
            pltpu.async_copy(bm1, g_h.at[rows, pl.ds(_GC_MV + 1 * 128, 128)], sem),
            pltpu.async_copy(bm2, g_h.at[rows, pl.ds(_GC_MV + 2 * 128, 128)], sem),
            pltpu.async_copy(bm3, g_h.at[rows, pl.ds(_GC_MV + 3 * 128, 128)], sem),
            pltpu.async_copy(bit, g_h.at[rows, pl.ds(_GC_IT, 128)], sem),
            pltpu.async_copy(bab, g_h.at[rows, pl.ds(_GC_AB, 128)], sem),
        ]
        for c in stores:
            c.wait()


@jax.jit
def _gather(idx_all, sp_t, mv_t, it_tp, ab_tp):
    mesh = plsc.VectorSubcoreMesh(core_axis_name="c", subcore_axis_name="s")
    return pl.kernel(
        _gather_body,
        out_type=jax.ShapeDtypeStruct((_NSLOT * _B, _GW), jnp.float32),
        mesh=mesh,
        scratch_types=[
            pltpu.VMEM((84, _NB), jnp.int32),
            pltpu.VMEM((_NB, 128), jnp.float32),
            pltpu.VMEM((_NB, 128), jnp.float32),
            pltpu.VMEM((_NB, 128), jnp.float32),
            pltpu.VMEM((_NB, 128), jnp.float32),
            pltpu.VMEM((_NB, 128), jnp.float32),
            pltpu.VMEM((_NB, 128), jnp.float32),
            pltpu.VMEM((_NB, 128), jnp.float32),
            pltpu.SemaphoreType.DMA,
        ],
    )(idx_all, sp_t, mv_t, it_tp, ab_tp)


def _asm_body(g_ref, ma_ref, pa_ref, o_ref):
    x = g_ref[...]
    ii = lax.broadcasted_iota(jnp.int32, (_NB, _NB), 0)
    jj = lax.broadcasted_iota(jnp.int32, (_NB, _NB), 1)
    eye = (ii == jj).astype(jnp.float32)
    y = lax.dot_general(x, eye, (((0,), (0,)), ((), ())),
                        preferred_element_type=jnp.float32)
    o_ref[0, 0, pl.ds(0, 16), 0] = y[0:128].reshape(16, 8, 128)
    o_ref[0, 0, pl.ds(16, 8), 0] = y[640:704].reshape(8, 8, 128)
    o_ref[0, 0, pl.ds(24, 8), 0] = y[768:832].reshape(8, 8, 128)
    o_ref[0, 0, pl.ds(32, 64), 0] = y[128:640].reshape(64, 8, 128)
    o_ref[0, 0, pl.ds(96, 4), 0] = ma_ref[0, 0, :, 0]
    o_ref[0, 0, pl.ds(100, 6), 0] = pa_ref[0, 0, :, 0]


@jax.jit
def _assemble(g, ma_l, pa_l):
    return pl.pallas_call(
        _asm_body,
        grid=(_NSLOT, _NBT),
        in_specs=[
            pl.BlockSpec((_NB, _GW), lambda slot, bt: (slot * _NBT + bt, 0)),
            pl.BlockSpec((1, 1, 4, 1, 8, 128),
                         lambda slot, bt: (slot // 6, slot % 6, 0, bt, 0, 0)),
            pl.BlockSpec((1, 1, 6, 1, 8, 128),
                         lambda slot, bt: (slot // 6, slot % 6, 0, bt, 0, 0)),
        ],
        out_specs=pl.BlockSpec((1, 1, _NCT, 1, 8, 128),
                               lambda slot, bt: (slot // 6, slot % 6, 0, bt, 0, 0)),
        out_shape=jax.ShapeDtypeStruct((2, 6, _NCT, _NBT, 8, 128), jnp.float32),
    )(g, ma_l, pa_l)


def kernel(fields, sides, species, moves, items, abilities, move_attributes,
           pokemon_attributes, species_table, item_table, ability_table,
           move_table):
    sp_i = jnp.transpose(species, (2, 1, 0)).reshape(12, _B)
    it_i = jnp.transpose(items, (2, 1, 0)).reshape(12, _B)
    ab_i = jnp.transpose(abilities, (2, 1, 0)).reshape(12, _B)
    mv_i = jnp.transpose(moves, (1, 2, 3, 0)).reshape(48, _B)
    idx_all = jnp.concatenate([sp_i, it_i, ab_i, mv_i], axis=0)
    it_tp = jnp.pad(item_table, ((0, 0), (0, 64)))
    ab_tp = jnp.pad(ability_table, ((0, 0), (0, 64)))
    g = _gather(idx_all, species_table, move_table, it_tp, ab_tp)
    ma_l = (move_attributes.transpose(1, 2, 3, 0, 4)
            .reshape(2, 6, 4, _NBT, 128, 8).transpose(0, 1, 2, 3, 5, 4))
    pa_l = (pokemon_attributes.transpose(1, 2, 3, 0)
            .reshape(2, 6, 6, 8, _NBT, 128).transpose(0, 1, 2, 4, 3, 5))
    out6 = _assemble(g, ma_l, pa_l)
    out = out6.transpose(3, 5, 0, 1, 2, 4).reshape(_B, 2, 6, _ROW)
    return (fields, sides, out)


# speedup vs baseline: 4.1973x; 1.2930x over previous
import jax
import jax.numpy as jnp
from jax import lax
from jax.experimental import pallas as pl
from jax.experimental.pallas import tpu as pltpu
from jax.experimental.pallas import tpu_sc as plsc

_B = 4096
_NW = 32
_NB = _B // _NW
_NBT = _B // 128
_NSLOT = 12
_ROW = 848
_NCT = _ROW // 8

_GW = 896
_GC_SP = 0
_GC_MV = 128
_GC_IT = 640
_GC_AB = 768

_MA_DIM = 32
_PA_DIM = 48


def _gather_body(idx_h, sp_t, mv_t, it_t, ab_t, g_h,
                 idx_v, bsp, bm0, bm1, bm2, bm3, bit, bab, sem):
    wid = lax.axis_index("s") * 2 + lax.axis_index("c")
    b0 = wid * _NB

    pltpu.sync_copy(idx_h.at[:, pl.ds(b0, _NB)], idx_v)

    for slot in range(_NSLOT):
        s, p = slot // 6, slot % 6
        q_sp = p * 2 + s
        gathers = [
            pltpu.async_copy(sp_t.at[idx_v.at[q_sp]], bsp, sem),
            pltpu.async_copy(it_t.at[idx_v.at[12 + q_sp]], bit, sem),
            pltpu.async_copy(ab_t.at[idx_v.at[24 + q_sp]], bab, sem),
            pltpu.async_copy(mv_t.at[idx_v.at[36 + slot * 4 + 0]], bm0, sem),
            pltpu.async_copy(mv_t.at[idx_v.at[36 + slot * 4 + 1]], bm1, sem),
            pltpu.async_copy(mv_t.at[idx_v.at[36 + slot * 4 + 2]], bm2, sem),
            pltpu.async_copy(mv_t.at[idx_v.at[36 + slot * 4 + 3]], bm3, sem),
        ]
        for c in gathers:
            c.wait()
        rows = pl.ds(slot * _B + b0, _NB)
        stores = [
            pltpu.async_copy(bsp, g_h.at[rows, pl.ds(_GC_SP, 128)], sem),
            pltpu.async_copy(bm0, g_h.at[rows, pl.ds(_GC_MV + 0 * 128, 128)], sem),
            pltpu.async_copy(bm1, g_h.at[rows, pl.ds(_GC_MV + 1 * 128, 128)], sem),
            pltpu.async_copy(bm2, g_h.at[rows, pl.ds(_GC_MV + 2 * 128, 128)], sem),
            pltpu.async_copy(bm3, g_h.at[rows, pl.ds(_GC_MV + 3 * 128, 128)], sem),
            pltpu.async_copy(bit, g_h.at[rows, pl.ds(_GC_IT, 128)], sem),
            pltpu.async_copy(bab, g_h.at[rows, pl.ds(_GC_AB, 128)], sem),
        ]
        for c in stores:
            c.wait()


@jax.jit
def _gather(idx_all, sp_t, mv_t, it_tp, ab_tp):
    mesh = plsc.VectorSubcoreMesh(core_axis_name="c", subcore_axis_name="s")
    return pl.kernel(
        _gather_body,
        out_type=jax.ShapeDtypeStruct((_NSLOT * _B, _GW), jnp.float32),
        mesh=mesh,
        scratch_types=[
            pltpu.VMEM((84, _NB), jnp.int32),
            pltpu.VMEM((_NB, 128), jnp.float32),
            pltpu.VMEM((_NB, 128), jnp.float32),
            pltpu.VMEM((_NB, 128), jnp.float32),
            pltpu.VMEM((_NB, 128), jnp.float32),
            pltpu.VMEM((_NB, 128), jnp.float32),
            pltpu.VMEM((_NB, 128), jnp.float32),
            pltpu.VMEM((_NB, 128), jnp.float32),
            pltpu.SemaphoreType.DMA,
        ],
    )(idx_all, sp_t, mv_t, it_tp, ab_tp)


_BTG = 2


def _asm_body(g_ref, ma_ref, pa_ref, o_ref):
    for k in range(_BTG):
        x = g_ref[pl.ds(k * _NB, _NB), :]
        y = jnp.transpose(x)
        o_ref[0, 0, pl.ds(0, 16), k] = y[0:128].reshape(16, 8, 128)
        o_ref[0, 0, pl.ds(16, 8), k] = y[640:704].reshape(8, 8, 128)
        o_ref[0, 0, pl.ds(24, 8), k] = y[768:832].reshape(8, 8, 128)
        o_ref[0, 0, pl.ds(32, 64), k] = y[128:640].reshape(64, 8, 128)
        o_ref[0, 0, pl.ds(96, 4), k] = ma_ref[0, 0, :, k]
        o_ref[0, 0, pl.ds(100, 6), k] = pa_ref[0, 0, :, k]


@jax.jit
def _assemble(g, ma_l, pa_l):
    return pl.pallas_call(
        _asm_body,
        grid=(_NSLOT, _NBT // _BTG),
        in_specs=[
            pl.BlockSpec((_NB * _BTG, _GW),
                         lambda slot, bt: (slot * (_NBT // _BTG) + bt, 0)),
            pl.BlockSpec((1, 1, 4, _BTG, 8, 128),
                         lambda slot, bt: (slot // 6, slot % 6, 0, bt, 0, 0)),
            pl.BlockSpec((1, 1, 6, _BTG, 8, 128),
                         lambda slot, bt: (slot // 6, slot % 6, 0, bt, 0, 0)),
        ],
        out_specs=pl.BlockSpec((1, 1, _NCT, _BTG, 8, 128),
                               lambda slot, bt: (slot // 6, slot % 6, 0, bt, 0, 0)),
        out_shape=jax.ShapeDtypeStruct((2, 6, _NCT, _NBT, 8, 128), jnp.float32),
    )(g, ma_l, pa_l)


def kernel(fields, sides, species, moves, items, abilities, move_attributes,
           pokemon_attributes, species_table, item_table, ability_table,
           move_table):
    sp_i = jnp.transpose(species, (2, 1, 0)).reshape(12, _B)
    it_i = jnp.transpose(items, (2, 1, 0)).reshape(12, _B)
    ab_i = jnp.transpose(abilities, (2, 1, 0)).reshape(12, _B)
    mv_i = jnp.transpose(moves, (1, 2, 3, 0)).reshape(48, _B)
    idx_all = jnp.concatenate([sp_i, it_i, ab_i, mv_i], axis=0)
    it_tp = jnp.pad(item_table, ((0, 0), (0, 64)))
    ab_tp = jnp.pad(ability_table, ((0, 0), (0, 64)))
    g = _gather(idx_all, species_table, move_table, it_tp, ab_tp)
    ma_l = (move_attributes.transpose(1, 2, 3, 0, 4)
            .reshape(2, 6, 4, _NBT, 128, 8).transpose(0, 1, 2, 3, 5, 4))
    pa_l = (pokemon_attributes.transpose(1, 2, 3, 0)
            .reshape(2, 6, 6, 8, _NBT, 128).transpose(0, 1, 2, 4, 3, 5))
    out6 = _assemble(g, ma_l, pa_l)
    out = out6.transpose(3, 5, 0, 1, 2, 4).reshape(_B, 2, 6, _ROW)
    return (fields, sides, out)


# speedup vs baseline: 4.2623x; 1.0155x over previous
import jax
import jax.numpy as jnp
from jax import lax
from jax.experimental import pallas as pl
from jax.experimental.pallas import tpu as pltpu
from jax.experimental.pallas import tpu_sc as plsc

_B = 4096
_NW = 32
_NB = _B // _NW
_NBT = _B // 128
_NSLOT = 12
_ROW = 848
_NCT = _ROW // 8

_GW = 896
_GC_SP = 0
_GC_MV = 128
_GC_IT = 640
_GC_AB = 768

_MA_DIM = 32
_PA_DIM = 48


_CB = 64

_GCOLS = (_GC_SP, _GC_MV, _GC_MV + 128, _GC_MV + 256, _GC_MV + 384,
          _GC_IT, _GC_AB)


def _gather_body(idx_h, sp_t, mv_t, it_t, ab_t, g_h, idx_v, *refs):
    bufs = (refs[0:7], refs[7:14])
    sems = (refs[14], refs[15])
    wid = lax.axis_index("s") * 2 + lax.axis_index("c")
    b0 = wid * _NB

    pltpu.sync_copy(idx_h.at[:, pl.ds(b0, _NB)], idx_v)

    def drain(par, gslot, ghalf):
        rows = pl.ds(gslot * _B + b0 + ghalf * _CB, _CB)
        for j in range(7):
            pltpu.make_async_copy(
                bufs[par][j], g_h.at[rows, pl.ds(_GCOLS[j], 128)],
                sems[par]).wait()

    for c in range(_NSLOT * 2):
        slot, half = c // 2, c % 2
        par = c % 2
        s, p = slot // 6, slot % 6
        q_sp = p * 2 + s
        if c >= 2:
            pc = c - 2
            drain(par, pc // 2, pc % 2)
        bsl = pl.ds(half * _CB, _CB)
        tabs = (sp_t, mv_t, mv_t, mv_t, mv_t, it_t, ab_t)
        qrows = (q_sp, 36 + slot * 4, 37 + slot * 4, 38 + slot * 4,
                 39 + slot * 4, 12 + q_sp, 24 + q_sp)
        gathers = [
            pltpu.async_copy(tabs[j].at[idx_v.at[qrows[j], bsl]],
                             bufs[par][j], sems[par])
            for j in range(7)
        ]
        for g in gathers:
            g.wait()
        rows = pl.ds(slot * _B + b0 + half * _CB, _CB)
        for j in range(7):
            pltpu.async_copy(bufs[par][j],
                             g_h.at[rows, pl.ds(_GCOLS[j], 128)], sems[par])
    drain(0, 11, 0)
    drain(1, 11, 1)


@jax.jit
def _gather(idx_all, sp_t, mv_t, it_tp, ab_tp):
    mesh = plsc.VectorSubcoreMesh(core_axis_name="c", subcore_axis_name="s")
    return pl.kernel(
        _gather_body,
        out_type=jax.ShapeDtypeStruct((_NSLOT * _B, _GW), jnp.float32),
        mesh=mesh,
        scratch_types=[pltpu.VMEM((84, _NB), jnp.int32)]
        + [pltpu.VMEM((_CB, 128), jnp.float32) for _ in range(14)]
        + [pltpu.SemaphoreType.DMA, pltpu.SemaphoreType.DMA],
    )(idx_all, sp_t, mv_t, it_tp, ab_tp)


_BTG = 2


def _asm_body(g_ref, ma_ref, pa_ref, o_ref):
    for k in range(_BTG):
        x = g_ref[pl.ds(k * _NB, _NB), :]
        y = jnp.transpose(x)
        o_ref[0, 0, pl.ds(0, 16), k] = y[0:128].reshape(16, 8, 128)
        o_ref[0, 0, pl.ds(16, 8), k] = y[640:704].reshape(8, 8, 128)
        o_ref[0, 0, pl.ds(24, 8), k] = y[768:832].reshape(8, 8, 128)
        o_ref[0, 0, pl.ds(32, 64), k] = y[128:640].reshape(64, 8, 128)
        o_ref[0, 0, pl.ds(96, 4), k] = ma_ref[0, 0, :, k]
        o_ref[0, 0, pl.ds(100, 6), k] = pa_ref[0, 0, :, k]


@jax.jit
def _assemble(g, ma_l, pa_l):
    return pl.pallas_call(
        _asm_body,
        grid=(_NSLOT, _NBT // _BTG),
        in_specs=[
            pl.BlockSpec((_NB * _BTG, _GW),
                         lambda slot, bt: (slot * (_NBT // _BTG) + bt, 0)),
            pl.BlockSpec((1, 1, 4, _BTG, 8, 128),
                         lambda slot, bt: (slot // 6, slot % 6, 0, bt, 0, 0)),
            pl.BlockSpec((1, 1, 6, _BTG, 8, 128),
                         lambda slot, bt: (slot // 6, slot % 6, 0, bt, 0, 0)),
        ],
        out_specs=pl.BlockSpec((1, 1, _NCT, _BTG, 8, 128),
                               lambda slot, bt: (slot // 6, slot % 6, 0, bt, 0, 0)),
        out_shape=jax.ShapeDtypeStruct((2, 6, _NCT, _NBT, 8, 128), jnp.float32),
    )(g, ma_l, pa_l)


def kernel(fields, sides, species, moves, items, abilities, move_attributes,
           pokemon_attributes, species_table, item_table, ability_table,
           move_table):
    sp_i = jnp.transpose(species, (2, 1, 0)).reshape(12, _B)
    it_i = jnp.transpose(items, (2, 1, 0)).reshape(12, _B)
    ab_i = jnp.transpose(abilities, (2, 1, 0)).reshape(12, _B)
    mv_i = jnp.transpose(moves, (1, 2, 3, 0)).reshape(48, _B)
    idx_all = jnp.concatenate([sp_i, it_i, ab_i, mv_i], axis=0)
    it_tp = jnp.pad(item_table, ((0, 0), (0, 64)))
    ab_tp = jnp.pad(ability_table, ((0, 0), (0, 64)))
    g = _gather(idx_all, species_table, move_table, it_tp, ab_tp)
    ma_l = (move_attributes.transpose(1, 2, 3, 0, 4)
            .reshape(2, 6, 4, _NBT, 128, 8).transpose(0, 1, 2, 3, 5, 4))
    pa_l = (pokemon_attributes.transpose(1, 2, 3, 0)
            .reshape(2, 6, 6, 8, _NBT, 128).transpose(0, 1, 2, 4, 3, 5))
    out6 = _assemble(g, ma_l, pa_l)
    out = out6.transpose(3, 5, 0, 1, 2, 4).reshape(_B, 2, 6, _ROW)
    return (fields, sides, out)


# speedup vs baseline: 4.5345x; 1.0639x over previous
import functools

import jax
import jax.numpy as jnp
from jax import lax
from jax.experimental import pallas as pl
from jax.experimental.pallas import tpu as pltpu
from jax.experimental.pallas import tpu_sc as plsc

_B = 4096
_NW = 32
_NBT = _B // 128
_NSLOT = 12
_ROW = 848
_NCT = _ROW // 8

_HB = _B // 2

_GW = 896
_GC_SP = 0
_GC_MV = 128
_GC_IT = 640
_GC_AB = 768
_GCOLS = (_GC_SP, _GC_MV, _GC_MV + 128, _GC_MV + 256, _GC_MV + 384,
          _GC_IT, _GC_AB)

_MA_DIM = 32
_PA_DIM = 48

_BTG = 2


def _gather_body(idx_h, sp_t, mv_t, it_t, ab_t, g_h, idx_v, *refs):
    nbw = _HB // _NW
    cb = nbw // 2
    bufs = (refs[0:7], refs[7:14])
    sems = (refs[14], refs[15])
    wid = lax.axis_index("s") * 2 + lax.axis_index("c")
    b0 = wid * nbw
    boff = (wid % 2) * nbw
    pltpu.sync_copy(idx_h.at[:, pl.ds((wid // 2) * 128, 128)], idx_v)

    def drain(par, gslot, ghalf):
        rows = pl.ds(gslot * _HB + b0 + ghalf * cb, cb)
        for j in range(7):
            pltpu.make_async_copy(
                bufs[par][j], g_h.at[rows, pl.ds(_GCOLS[j], 128)],
                sems[par]).wait()

    for c in range(_NSLOT * 2):
        slot, half = c // 2, c % 2
        par = c % 2
        s, p = slot // 6, slot % 6
        q_sp = p * 2 + s
        if c >= 2:
            pc = c - 2
            drain(par, pc // 2, pc % 2)
        bsl = pl.ds(boff + half * cb, cb)
        tabs = (sp_t, mv_t, mv_t, mv_t, mv_t, it_t, ab_t)
        qrows = (q_sp, 36 + slot * 4, 37 + slot * 4, 38 + slot * 4,
                 39 + slot * 4, 12 + q_sp, 24 + q_sp)
        gathers = [
            pltpu.async_copy(tabs[j].at[idx_v.at[qrows[j], bsl]],
                             bufs[par][j], sems[par])
            for j in range(7)
        ]
        for g in gathers:
            g.wait()
        rows = pl.ds(slot * _HB + b0 + half * cb, cb)
        for j in range(7):
            pltpu.async_copy(bufs[par][j],
                             g_h.at[rows, pl.ds(_GCOLS[j], 128)], sems[par])
    drain(0, 11, 0)
    drain(1, 11, 1)


@jax.jit
def _gather(idx_half, sp_t, mv_t, it_tp, ab_tp):
    nbw = _HB // _NW
    cb = nbw // 2
    mesh = plsc.VectorSubcoreMesh(core_axis_name="c", subcore_axis_name="s")
    return pl.kernel(
        _gather_body,
        out_type=jax.ShapeDtypeStruct((_NSLOT * _HB, _GW), jnp.float32),
        mesh=mesh,
        scratch_types=[pltpu.VMEM((84, 128), jnp.int32)]
        + [pltpu.VMEM((cb, 128), jnp.float32) for _ in range(14)]
        + [pltpu.SemaphoreType.DMA, pltpu.SemaphoreType.DMA],
    )(idx_half, sp_t, mv_t, it_tp, ab_tp)


def _asm_body(g_ref, ma_ref, pa_ref, *rest):
    o_ref = rest[-1]
    for k in range(_BTG):
        x = g_ref[pl.ds(k * 128, 128), :]
        y = jnp.transpose(x)
        o_ref[0, 0, pl.ds(0, 16), k] = y[0:128].reshape(16, 8, 128)
        o_ref[0, 0, pl.ds(16, 8), k] = y[640:704].reshape(8, 8, 128)
        o_ref[0, 0, pl.ds(24, 8), k] = y[768:832].reshape(8, 8, 128)
        o_ref[0, 0, pl.ds(32, 64), k] = y[128:640].reshape(64, 8, 128)
        o_ref[0, 0, pl.ds(96, 4), k] = ma_ref[0, 0, :, k]
        o_ref[0, 0, pl.ds(100, 6), k] = pa_ref[0, 0, :, k]


_HBT = _HB // 128 // _BTG


def _mk_specs(hh):
    in_specs = [
        pl.BlockSpec((128 * _BTG, _GW), lambda slot, bt: (slot * _HBT + bt, 0)),
        pl.BlockSpec((1, 1, 4, _BTG, 8, 128),
                     lambda slot, bt: (slot // 6, slot % 6, 0, hh * _HBT + bt, 0, 0)),
        pl.BlockSpec((1, 1, 6, _BTG, 8, 128),
                     lambda slot, bt: (slot // 6, slot % 6, 0, hh * _HBT + bt, 0, 0)),
    ]
    out_spec = pl.BlockSpec(
        (1, 1, _NCT, _BTG, 8, 128),
        lambda slot, bt: (slot // 6, slot % 6, 0, hh * _HBT + bt, 0, 0))
    return in_specs, out_spec


_OUT6 = jax.ShapeDtypeStruct((2, 6, _NCT, _NBT, 8, 128), jnp.float32)


@jax.jit
def _assemble0(g, ma_l, pa_l):
    in_specs, out_spec = _mk_specs(0)
    return pl.pallas_call(
        _asm_body, grid=(_NSLOT, _HBT), in_specs=in_specs,
        out_specs=out_spec, out_shape=_OUT6,
    )(g, ma_l, pa_l)


@jax.jit
def _assemble1(g, ma_l, pa_l, prev):
    in_specs, out_spec = _mk_specs(1)
    in_specs.append(pl.BlockSpec(memory_space=pl.ANY))
    return pl.pallas_call(
        _asm_body, grid=(_NSLOT, _HBT), in_specs=in_specs,
        out_specs=out_spec, out_shape=_OUT6,
        input_output_aliases={3: 0},
    )(g, ma_l, pa_l, prev)


def kernel(fields, sides, species, moves, items, abilities, move_attributes,
           pokemon_attributes, species_table, item_table, ability_table,
           move_table):
    sp_i = jnp.transpose(species, (2, 1, 0)).reshape(12, _B)
    it_i = jnp.transpose(items, (2, 1, 0)).reshape(12, _B)
    ab_i = jnp.transpose(abilities, (2, 1, 0)).reshape(12, _B)
    mv_i = jnp.transpose(moves, (1, 2, 3, 0)).reshape(48, _B)
    idx_all = jnp.concatenate([sp_i, it_i, ab_i, mv_i], axis=0)
    it_tp = jnp.pad(item_table, ((0, 0), (0, 64)))
    ab_tp = jnp.pad(ability_table, ((0, 0), (0, 64)))
    ma_l = (move_attributes.transpose(1, 2, 3, 0, 4)
            .reshape(2, 6, 4, _NBT, 128, 8).transpose(0, 1, 2, 3, 5, 4))
    pa_l = (pokemon_attributes.transpose(1, 2, 3, 0)
            .reshape(2, 6, 6, 8, _NBT, 128).transpose(0, 1, 2, 4, 3, 5))
    g0 = _gather(idx_all[:, :_HB], species_table, move_table, it_tp, ab_tp)
    o0 = _assemble0(g0, ma_l, pa_l)
    g1 = _gather(idx_all[:, _HB:], species_table, move_table, it_tp, ab_tp)
    o1 = _assemble1(g1, ma_l, pa_l, o0)
    out = o1.transpose(3, 5, 0, 1, 2, 4).reshape(_B, 2, 6, _ROW)
    return (fields, sides, out)


# speedup vs baseline: 4.6845x; 1.0331x over previous
import functools

import jax
import jax.numpy as jnp
from jax import lax
from jax.experimental import pallas as pl
from jax.experimental.pallas import tpu as pltpu
from jax.experimental.pallas import tpu_sc as plsc

_B = 4096
_NW = 32
_NBT = _B // 128
_NSLOT = 12
_ROW = 848
_NCT = _ROW // 8

_NP = 4
_HB = _B // _NP

_GW = 896
_GC_SP = 0
_GC_MV = 128
_GC_IT = 640
_GC_AB = 768
_GCOLS = (_GC_SP, _GC_MV, _GC_MV + 128, _GC_MV + 256, _GC_MV + 384,
          _GC_IT, _GC_AB)

_MA_DIM = 32
_PA_DIM = 48

_BTG = 2


def _gather_body(idx_h, sp_t, mv_t, it_t, ab_t, g_h, idx_v, *refs):
    nbw = _HB // _NW
    cb = nbw // 2
    bufs = (refs[0:7], refs[7:14])
    sems = (refs[14], refs[15])
    wid = lax.axis_index("s") * 2 + lax.axis_index("c")
    b0 = wid * nbw
    wpt = 128 // nbw
    boff = (wid % wpt) * nbw
    pltpu.sync_copy(idx_h.at[:, pl.ds((wid // wpt) * 128, 128)], idx_v)

    def drain(par, gslot, ghalf):
        rows = pl.ds(gslot * _HB + b0 + ghalf * cb, cb)
        for j in range(7):
            pltpu.make_async_copy(
                bufs[par][j], g_h.at[rows, pl.ds(_GCOLS[j], 128)],
                sems[par]).wait()

    for c in range(_NSLOT * 2):
        slot, half = c // 2, c % 2
        par = c % 2
        s, p = slot // 6, slot % 6
        q_sp = p * 2 + s
        if c >= 2:
            pc = c - 2
            drain(par, pc // 2, pc % 2)
        bsl = pl.ds(boff + half * cb, cb)
        tabs = (sp_t, mv_t, mv_t, mv_t, mv_t, it_t, ab_t)
        qrows = (q_sp, 36 + slot * 4, 37 + slot * 4, 38 + slot * 4,
                 39 + slot * 4, 12 + q_sp, 24 + q_sp)
        gathers = [
            pltpu.async_copy(tabs[j].at[idx_v.at[qrows[j], bsl]],
                             bufs[par][j], sems[par])
            for j in range(7)
        ]
        for g in gathers:
            g.wait()
        rows = pl.ds(slot * _HB + b0 + half * cb, cb)
        for j in range(7):
            pltpu.async_copy(bufs[par][j],
                             g_h.at[rows, pl.ds(_GCOLS[j], 128)], sems[par])
    drain(0, 11, 0)
    drain(1, 11, 1)


@jax.jit
def _gather(idx_half, sp_t, mv_t, it_tp, ab_tp):
    nbw = _HB // _NW
    cb = nbw // 2
    mesh = plsc.VectorSubcoreMesh(core_axis_name="c", subcore_axis_name="s")
    return pl.kernel(
        _gather_body,
        out_type=jax.ShapeDtypeStruct((_NSLOT * _HB, _GW), jnp.float32),
        mesh=mesh,
        scratch_types=[pltpu.VMEM((84, 128), jnp.int32)]
        + [pltpu.VMEM((cb, 128), jnp.float32) for _ in range(14)]
        + [pltpu.SemaphoreType.DMA, pltpu.SemaphoreType.DMA],
    )(idx_half, sp_t, mv_t, it_tp, ab_tp)


def _asm_body(g_ref, ma_ref, pa_ref, *rest):
    o_ref = rest[-1]
    for k in range(_BTG):
        x = g_ref[pl.ds(k * 128, 128), :]
        y = jnp.transpose(x)
        o_ref[0, 0, pl.ds(0, 16), k] = y[0:128].reshape(16, 8, 128)
        o_ref[0, 0, pl.ds(16, 8), k] = y[640:704].reshape(8, 8, 128)
        o_ref[0, 0, pl.ds(24, 8), k] = y[768:832].reshape(8, 8, 128)
        o_ref[0, 0, pl.ds(32, 64), k] = y[128:640].reshape(64, 8, 128)
        o_ref[0, 0, pl.ds(96, 4), k] = ma_ref[0, 0, :, k]
        o_ref[0, 0, pl.ds(100, 6), k] = pa_ref[0, 0, :, k]


_HBT = _HB // 128 // _BTG


def _mk_specs(hh):
    in_specs = [
        pl.BlockSpec((128 * _BTG, _GW), lambda slot, bt: (slot * _HBT + bt, 0)),
        pl.BlockSpec((1, 1, 4, _BTG, 8, 128),
                     lambda slot, bt: (slot // 6, slot % 6, 0, hh * _HBT + bt, 0, 0)),
        pl.BlockSpec((1, 1, 6, _BTG, 8, 128),
                     lambda slot, bt: (slot // 6, slot % 6, 0, hh * _HBT + bt, 0, 0)),
    ]
    out_spec = pl.BlockSpec(
        (1, 1, _NCT, _BTG, 8, 128),
        lambda slot, bt: (slot // 6, slot % 6, 0, hh * _HBT + bt, 0, 0))
    return in_specs, out_spec


_OUT6 = jax.ShapeDtypeStruct((2, 6, _NCT, _NBT, 8, 128), jnp.float32)


@jax.jit
def _assemble_first(g, ma_l, pa_l):
    in_specs, out_spec = _mk_specs(0)
    return pl.pallas_call(
        _asm_body, grid=(_NSLOT, _HBT), in_specs=in_specs,
        out_specs=out_spec, out_shape=_OUT6,
    )(g, ma_l, pa_l)


@functools.partial(jax.jit, static_argnames=("part",))
def _assemble_next(g, ma_l, pa_l, prev, part):
    in_specs, out_spec = _mk_specs(part)
    in_specs.append(pl.BlockSpec(memory_space=pl.ANY))
    return pl.pallas_call(
        _asm_body, grid=(_NSLOT, _HBT), in_specs=in_specs,
        out_specs=out_spec, out_shape=_OUT6,
        input_output_aliases={3: 0},
    )(g, ma_l, pa_l, prev)


def kernel(fields, sides, species, moves, items, abilities, move_attributes,
           pokemon_attributes, species_table, item_table, ability_table,
           move_table):
    sp_i = jnp.transpose(species, (2, 1, 0)).reshape(12, _B)
    it_i = jnp.transpose(items, (2, 1, 0)).reshape(12, _B)
    ab_i = jnp.transpose(abilities, (2, 1, 0)).reshape(12, _B)
    mv_i = jnp.transpose(moves, (1, 2, 3, 0)).reshape(48, _B)
    idx_all = jnp.concatenate([sp_i, it_i, ab_i, mv_i], axis=0)
    it_tp = jnp.pad(item_table, ((0, 0), (0, 64)))
    ab_tp = jnp.pad(ability_table, ((0, 0), (0, 64)))
    ma_l = (move_attributes.transpose(1, 2, 3, 0, 4)
            .reshape(2, 6, 4, _NBT, 128, 8).transpose(0, 1, 2, 3, 5, 4))
    pa_l = (pokemon_attributes.transpose(1, 2, 3, 0)
            .reshape(2, 6, 6, 8, _NBT, 128).transpose(0, 1, 2, 4, 3, 5))
    o = None
    for part in range(_NP):
        idx_p = lax.slice_in_dim(idx_all, part * _HB, (part + 1) * _HB, axis=1)
        g = _gather(idx_p, species_table, move_table, it_tp, ab_tp)
        if part == 0:
            o = _assemble_first(g, ma_l, pa_l)
        else:
            o = _assemble_next(g, ma_l, pa_l, o, part=part)
    out = o.transpose(3, 5, 0, 1, 2, 4).reshape(_B, 2, 6, _ROW)
    return (fields, sides, out)


# speedup vs baseline: 4.9110x; 1.0483x over previous
import functools

import jax
import jax.numpy as jnp
from jax import lax
from jax.experimental import pallas as pl
from jax.experimental.pallas import tpu as pltpu
from jax.experimental.pallas import tpu_sc as plsc

_B = 4096
_NW = 32
_NBT = _B // 128
_NSLOT = 12
_ROW = 848
_NCT = _ROW // 8

_NP = 4
_HB = _B // _NP

_GW = 896
_GC_SP = 0
_GC_MV = 128
_GC_IT = 640
_GC_AB = 768
_GCOLS = (_GC_SP, _GC_MV, _GC_MV + 128, _GC_MV + 256, _GC_MV + 384,
          _GC_IT, _GC_AB)

_MA_DIM = 32
_PA_DIM = 48

_BTG = 4


def _gather_body(idx_h, sp_t, mv_t, it_t, ab_t, g_h, idx_v, *refs):
    nbw = _HB // _NW
    cb = nbw // 2
    bufs = (refs[0:7], refs[7:14])
    sems = (refs[14], refs[15])
    wid = lax.axis_index("s") * 2 + lax.axis_index("c")
    b0 = wid * nbw
    wpt = 128 // nbw
    boff = (wid % wpt) * nbw
    pltpu.sync_copy(idx_h.at[:, pl.ds((wid // wpt) * 128, 128)], idx_v)

    def drain(par, gslot, ghalf):
        rows = pl.ds(gslot * _HB + b0 + ghalf * cb, cb)
        for j in range(7):
            pltpu.make_async_copy(
                bufs[par][j], g_h.at[rows, pl.ds(_GCOLS[j], 128)],
                sems[par]).wait()

    for c in range(_NSLOT * 2):
        slot, half = c // 2, c % 2
        par = c % 2
        s, p = slot // 6, slot % 6
        q_sp = p * 2 + s
        if c >= 2:
            pc = c - 2
            drain(par, pc // 2, pc % 2)
        bsl = pl.ds(boff + half * cb, cb)
        tabs = (sp_t, mv_t, mv_t, mv_t, mv_t, it_t, ab_t)
        qrows = (q_sp, 36 + slot * 4, 37 + slot * 4, 38 + slot * 4,
                 39 + slot * 4, 12 + q_sp, 24 + q_sp)
        gathers = [
            pltpu.async_copy(tabs[j].at[idx_v.at[qrows[j], bsl]],
                             bufs[par][j], sems[par])
            for j in range(7)
        ]
        for g in gathers:
            g.wait()
        rows = pl.ds(slot * _HB + b0 + half * cb, cb)
        for j in range(7):
            pltpu.async_copy(bufs[par][j],
                             g_h.at[rows, pl.ds(_GCOLS[j], 128)], sems[par])
    drain(0, 11, 0)
    drain(1, 11, 1)


@jax.jit
def _gather(idx_half, sp_t, mv_t, it_tp, ab_tp):
    nbw = _HB // _NW
    cb = nbw // 2
    mesh = plsc.VectorSubcoreMesh(core_axis_name="c", subcore_axis_name="s")
    return pl.kernel(
        _gather_body,
        out_type=jax.ShapeDtypeStruct((_NSLOT * _HB, _GW), jnp.float32),
        mesh=mesh,
        scratch_types=[pltpu.VMEM((84, 128), jnp.int32)]
        + [pltpu.VMEM((cb, 128), jnp.float32) for _ in range(14)]
        + [pltpu.SemaphoreType.DMA, pltpu.SemaphoreType.DMA],
    )(idx_half, sp_t, mv_t, it_tp, ab_tp)


def _asm_body(g_ref, ma_ref, pa_ref, *rest):
    o_ref = rest[-1]
    for k in range(_BTG):
        x = g_ref[pl.ds(k * 128, 128), :]
        y = jnp.transpose(x)
        o_ref[0, 0, pl.ds(0, 16), k] = y[0:128].reshape(16, 8, 128)
        o_ref[0, 0, pl.ds(16, 8), k] = y[640:704].reshape(8, 8, 128)
        o_ref[0, 0, pl.ds(24, 8), k] = y[768:832].reshape(8, 8, 128)
        o_ref[0, 0, pl.ds(32, 64), k] = y[128:640].reshape(64, 8, 128)
        o_ref[0, 0, pl.ds(96, 4), k] = ma_ref[0, 0, :, k]
        o_ref[0, 0, pl.ds(100, 6), k] = pa_ref[0, 0, :, k]


_HBT = _HB // 128 // _BTG


def _mk_specs(hh):
    in_specs = [
        pl.BlockSpec((128 * _BTG, _GW), lambda slot, bt: (slot * _HBT + bt, 0)),
        pl.BlockSpec((1, 1, 4, _BTG, 8, 128),
                     lambda slot, bt: (slot // 6, slot % 6, 0, hh * _HBT + bt, 0, 0)),
        pl.BlockSpec((1, 1, 6, _BTG, 8, 128),
                     lambda slot, bt: (slot // 6, slot % 6, 0, hh * _HBT + bt, 0, 0)),
    ]
    out_spec = pl.BlockSpec(
        (1, 1, _NCT, _BTG, 8, 128),
        lambda slot, bt: (slot // 6, slot % 6, 0, hh * _HBT + bt, 0, 0))
    return in_specs, out_spec


_OUT6 = jax.ShapeDtypeStruct((2, 6, _NCT, _NBT, 8, 128), jnp.float32)


@jax.jit
def _assemble_first(g, ma_l, pa_l):
    in_specs, out_spec = _mk_specs(0)
    return pl.pallas_call(
        _asm_body, grid=(_NSLOT, _HBT), in_specs=in_specs,
        out_specs=out_spec, out_shape=_OUT6,
    )(g, ma_l, pa_l)


@functools.partial(jax.jit, static_argnames=("part",))
def _assemble_next(g, ma_l, pa_l, prev, part):
    in_specs, out_spec = _mk_specs(part)
    in_specs.append(pl.BlockSpec(memory_space=pl.ANY))
    return pl.pallas_call(
        _asm_body, grid=(_NSLOT, _HBT), in_specs=in_specs,
        out_specs=out_spec, out_shape=_OUT6,
        input_output_aliases={3: 0},
    )(g, ma_l, pa_l, prev)


def kernel(fields, sides, species, moves, items, abilities, move_attributes,
           pokemon_attributes, species_table, item_table, ability_table,
           move_table):
    sp_i = jnp.transpose(species, (2, 1, 0)).reshape(12, _B)
    it_i = jnp.transpose(items, (2, 1, 0)).reshape(12, _B)
    ab_i = jnp.transpose(abilities, (2, 1, 0)).reshape(12, _B)
    mv_i = jnp.transpose(moves, (1, 2, 3, 0)).reshape(48, _B)
    idx_all = jnp.concatenate([sp_i, it_i, ab_i, mv_i], axis=0)
    it_tp = jnp.pad(item_table, ((0, 0), (0, 64)))
    ab_tp = jnp.pad(ability_table, ((0, 0), (0, 64)))
    ma_l = (move_attributes.transpose(1, 2, 3, 0, 4)
            .reshape(2, 6, 4, _NBT, 128, 8).transpose(0, 1, 2, 3, 5, 4))
    pa_l = (pokemon_attributes.transpose(1, 2, 3, 0)
            .reshape(2, 6, 6, 8, _NBT, 128).transpose(0, 1, 2, 4, 3, 5))
    o = None
    for part in range(_NP):
        idx_p = lax.slice_in_dim(idx_all, part * _HB, (part + 1) * _HB, axis=1)
        g = _gather(idx_p, species_table, move_table, it_tp, ab_tp)
        if part == 0:
            o = _assemble_first(g, ma_l, pa_l)
        else:
            o = _assemble_next(g, ma_l, pa_l, o, part=part)
    out = o.transpose(3, 5, 0, 1, 2, 4).reshape(_B, 2, 6, _ROW)
    return (fields, sides, out)


# speedup vs baseline: 5.0112x; 1.0204x over previous
import functools

import jax
import jax.numpy as jnp
from jax import lax
from jax.experimental import pallas as pl
from jax.experimental.pallas import tpu as pltpu
from jax.experimental.pallas import tpu_sc as plsc

_B = 4096
_NW = 32
_NBT = _B // 128
_NSLOT = 12
_ROW = 848
_NCT = _ROW // 8

_NP = 4
_HB = _B // _NP

_GW = 896
_GC_SP = 0
_GC_MV = 128
_GC_IT = 640
_GC_AB = 768
_GCOLS = (_GC_SP, _GC_MV, _GC_MV + 128, _GC_MV + 256, _GC_MV + 384,
          _GC_IT, _GC_AB)

_MA_DIM = 32
_PA_DIM = 48

_BTG = 8


def _gather_body(idx_h, sp_t, mv_t, it_t, ab_t, g_h, idx_v, *refs):
    nbw = _HB // _NW
    cb = nbw // 2
    bufs = (refs[0:7], refs[7:14])
    sems = (refs[14], refs[15])
    wid = lax.axis_index("s") * 2 + lax.axis_index("c")
    b0 = wid * nbw
    wpt = 128 // nbw
    boff = (wid % wpt) * nbw
    pltpu.sync_copy(idx_h.at[:, pl.ds((wid // wpt) * 128, 128)], idx_v)

    def drain(par, gslot, ghalf):
        rows = pl.ds(gslot * _HB + b0 + ghalf * cb, cb)
        for j in range(7):
            pltpu.make_async_copy(
                bufs[par][j], g_h.at[rows, pl.ds(_GCOLS[j], 128)],
                sems[par]).wait()

    for c in range(_NSLOT * 2):
        slot, half = c // 2, c % 2
        par = c % 2
        s, p = slot // 6, slot % 6
        q_sp = p * 2 + s
        if c >= 2:
            pc = c - 2
            drain(par, pc // 2, pc % 2)
        bsl = pl.ds(boff + half * cb, cb)
        tabs = (sp_t, mv_t, mv_t, mv_t, mv_t, it_t, ab_t)
        qrows = (q_sp, 36 + slot * 4, 37 + slot * 4, 38 + slot * 4,
                 39 + slot * 4, 12 + q_sp, 24 + q_sp)
        gathers = [
            pltpu.async_copy(tabs[j].at[idx_v.at[qrows[j], bsl]],
                             bufs[par][j], sems[par])
            for j in range(7)
        ]
        for g in gathers:
            g.wait()
        rows = pl.ds(slot * _HB + b0 + half * cb, cb)
        for j in range(7):
            pltpu.async_copy(bufs[par][j],
                             g_h.at[rows, pl.ds(_GCOLS[j], 128)], sems[par])
    drain(0, 11, 0)
    drain(1, 11, 1)


@jax.jit
def _gather(idx_half, sp_t, mv_t, it_tp, ab_tp):
    nbw = _HB // _NW
    cb = nbw // 2
    mesh = plsc.VectorSubcoreMesh(core_axis_name="c", subcore_axis_name="s")
    return pl.kernel(
        _gather_body,
        out_type=jax.ShapeDtypeStruct((_NSLOT * _HB, _GW), jnp.float32),
        mesh=mesh,
        scratch_types=[pltpu.VMEM((84, 128), jnp.int32)]
        + [pltpu.VMEM((cb, 128), jnp.float32) for _ in range(14)]
        + [pltpu.SemaphoreType.DMA, pltpu.SemaphoreType.DMA],
    )(idx_half, sp_t, mv_t, it_tp, ab_tp)


def _asm_body(g_ref, ma_ref, pa_ref, *rest):
    o_ref = rest[-1]
    for k in range(_BTG):
        x = g_ref[pl.ds(k * 128, 128), :]
        y = jnp.transpose(x)
        o_ref[0, 0, pl.ds(0, 16), k] = y[0:128].reshape(16, 8, 128)
        o_ref[0, 0, pl.ds(16, 8), k] = y[640:704].reshape(8, 8, 128)
        o_ref[0, 0, pl.ds(24, 8), k] = y[768:832].reshape(8, 8, 128)
        o_ref[0, 0, pl.ds(32, 64), k] = y[128:640].reshape(64, 8, 128)
        o_ref[0, 0, pl.ds(96, 4), k] = ma_ref[0, 0, :, k]
        o_ref[0, 0, pl.ds(100, 6), k] = pa_ref[0, 0, :, k]


_HBT = _HB // 128 // _BTG


def _mk_specs(hh):
    in_specs = [
        pl.BlockSpec((128 * _BTG, _GW), lambda slot, bt: (slot * _HBT + bt, 0)),
        pl.BlockSpec((1, 1, 4, _BTG, 8, 128),
                     lambda slot, bt: (slot // 6, slot % 6, 0, hh * _HBT + bt, 0, 0)),
        pl.BlockSpec((1, 1, 6, _BTG, 8, 128),
                     lambda slot, bt: (slot // 6, slot % 6, 0, hh * _HBT + bt, 0, 0)),
    ]
    out_spec = pl.BlockSpec(
        (1, 1, _NCT, _BTG, 8, 128),
        lambda slot, bt: (slot // 6, slot % 6, 0, hh * _HBT + bt, 0, 0))
    return in_specs, out_spec


_OUT6 = jax.ShapeDtypeStruct((2, 6, _NCT, _NBT, 8, 128), jnp.float32)


@jax.jit
def _assemble_first(g, ma_l, pa_l):
    in_specs, out_spec = _mk_specs(0)
    return pl.pallas_call(
        _asm_body, grid=(_NSLOT, _HBT), in_specs=in_specs,
        out_specs=out_spec, out_shape=_OUT6,
    )(g, ma_l, pa_l)


@functools.partial(jax.jit, static_argnames=("part",))
def _assemble_next(g, ma_l, pa_l, prev, part):
    in_specs, out_spec = _mk_specs(part)
    in_specs.append(pl.BlockSpec(memory_space=pl.ANY))
    return pl.pallas_call(
        _asm_body, grid=(_NSLOT, _HBT), in_specs=in_specs,
        out_specs=out_spec, out_shape=_OUT6,
        input_output_aliases={3: 0},
    )(g, ma_l, pa_l, prev)


def kernel(fields, sides, species, moves, items, abilities, move_attributes,
           pokemon_attributes, species_table, item_table, ability_table,
           move_table):
    sp_i = jnp.transpose(species, (2, 1, 0)).reshape(12, _B)
    it_i = jnp.transpose(items, (2, 1, 0)).reshape(12, _B)
    ab_i = jnp.transpose(abilities, (2, 1, 0)).reshape(12, _B)
    mv_i = jnp.transpose(moves, (1, 2, 3, 0)).reshape(48, _B)
    idx_all = jnp.concatenate([sp_i, it_i, ab_i, mv_i], axis=0)
    it_tp = jnp.pad(item_table, ((0, 0), (0, 64)))
    ab_tp = jnp.pad(ability_table, ((0, 0), (0, 64)))
    ma_l = (move_attributes.transpose(1, 2, 3, 0, 4)
            .reshape(2, 6, 4, _NBT, 128, 8).transpose(0, 1, 2, 3, 5, 4))
    pa_l = (pokemon_attributes.transpose(1, 2, 3, 0)
            .reshape(2, 6, 6, 8, _NBT, 128).transpose(0, 1, 2, 4, 3, 5))
    o = None
    for part in range(_NP):
        idx_p = lax.slice_in_dim(idx_all, part * _HB, (part + 1) * _HB, axis=1)
        g = _gather(idx_p, species_table, move_table, it_tp, ab_tp)
        if part == 0:
            o = _assemble_first(g, ma_l, pa_l)
        else:
            o = _assemble_next(g, ma_l, pa_l, o, part=part)
    out = o.transpose(3, 5, 0, 1, 2, 4).reshape(_B, 2, 6, _ROW)
    return (fields, sides, out)


# speedup vs baseline: 5.4102x; 1.0796x over previous
import functools

import jax
import jax.numpy as jnp
from jax import lax
from jax.experimental import pallas as pl
from jax.experimental.pallas import tpu as pltpu
from jax.experimental.pallas import tpu_sc as plsc

_B = 4096
_NW = 32
_NBT = _B // 128
_NSLOT = 12
_ROW = 848
_NCT = _ROW // 8

_NP = 2
_HB = _B // _NP

_GW = 896
_GC_SP = 0
_GC_MV = 128
_GC_IT = 640
_GC_AB = 768
_GCOLS = (_GC_SP, _GC_MV, _GC_MV + 128, _GC_MV + 256, _GC_MV + 384,
          _GC_IT, _GC_AB)

_MA_DIM = 32
_PA_DIM = 48

_BTG = 8


def _gather_body(idx_h, sp_t, mv_t, it_t, ab_t, g_h, idx_v, *refs):
    nbw = _HB // _NW
    cb = nbw // 2
    bufs = (refs[0:7], refs[7:14])
    sems = (refs[14], refs[15])
    wid = lax.axis_index("s") * 2 + lax.axis_index("c")
    b0 = wid * nbw
    wpt = 128 // nbw
    boff = (wid % wpt) * nbw
    pltpu.sync_copy(idx_h.at[:, pl.ds((wid // wpt) * 128, 128)], idx_v)

    def drain(par, gslot, ghalf):
        rows = pl.ds(gslot * _HB + b0 + ghalf * cb, cb)
        for j in range(7):
            pltpu.make_async_copy(
                bufs[par][j], g_h.at[rows, pl.ds(_GCOLS[j], 128)],
                sems[par]).wait()

    for c in range(_NSLOT * 2):
        slot, half = c // 2, c % 2
        par = c % 2
        s, p = slot // 6, slot % 6
        q_sp = p * 2 + s
        if c >= 2:
            pc = c - 2
            drain(par, pc // 2, pc % 2)
        bsl = pl.ds(boff + half * cb, cb)
        tabs = (sp_t, mv_t, mv_t, mv_t, mv_t, it_t, ab_t)
        qrows = (q_sp, 36 + slot * 4, 37 + slot * 4, 38 + slot * 4,
                 39 + slot * 4, 12 + q_sp, 24 + q_sp)
        gathers = [
            pltpu.async_copy(tabs[j].at[idx_v.at[qrows[j], bsl]],
                             bufs[par][j], sems[par])
            for j in range(7)
        ]
        for g in gathers:
            g.wait()
        rows = pl.ds(slot * _HB + b0 + half * cb, cb)
        for j in range(7):
            pltpu.async_copy(bufs[par][j],
                             g_h.at[rows, pl.ds(_GCOLS[j], 128)], sems[par])
    drain(0, 11, 0)
    drain(1, 11, 1)


@jax.jit
def _gather(idx_half, sp_t, mv_t, it_tp, ab_tp):
    nbw = _HB // _NW
    cb = nbw // 2
    mesh = plsc.VectorSubcoreMesh(core_axis_name="c", subcore_axis_name="s")
    return pl.kernel(
        _gather_body,
        out_type=jax.ShapeDtypeStruct((_NSLOT * _HB, _GW), jnp.float32),
        mesh=mesh,
        scratch_types=[pltpu.VMEM((84, 128), jnp.int32)]
        + [pltpu.VMEM((cb, 128), jnp.float32) for _ in range(14)]
        + [pltpu.SemaphoreType.DMA, pltpu.SemaphoreType.DMA],
    )(idx_half, sp_t, mv_t, it_tp, ab_tp)


def _asm_body(g_ref, ma_ref, pa_ref, *rest):
    o_ref = rest[-1]
    for k in range(_BTG):
        x = g_ref[pl.ds(k * 128, 128), :]
        y = jnp.transpose(x)
        o_ref[0, 0, pl.ds(0, 16), k] = y[0:128].reshape(16, 8, 128)
        o_ref[0, 0, pl.ds(16, 8), k] = y[640:704].reshape(8, 8, 128)
        o_ref[0, 0, pl.ds(24, 8), k] = y[768:832].reshape(8, 8, 128)
        o_ref[0, 0, pl.ds(32, 64), k] = y[128:640].reshape(64, 8, 128)
        o_ref[0, 0, pl.ds(96, 4), k] = ma_ref[0, 0, :, k]
        o_ref[0, 0, pl.ds(100, 6), k] = pa_ref[0, 0, :, k]


_HBT = _HB // 128 // _BTG


def _mk_specs(hh):
    in_specs = [
        pl.BlockSpec((128 * _BTG, _GW), lambda slot, bt: (slot * _HBT + bt, 0)),
        pl.BlockSpec((1, 1, 4, _BTG, 8, 128),
                     lambda slot, bt: (slot // 6, slot % 6, 0, hh * _HBT + bt, 0, 0)),
        pl.BlockSpec((1, 1, 6, _BTG, 8, 128),
                     lambda slot, bt: (slot // 6, slot % 6, 0, hh * _HBT + bt, 0, 0)),
    ]
    out_spec = pl.BlockSpec(
        (1, 1, _NCT, _BTG, 8, 128),
        lambda slot, bt: (slot // 6, slot % 6, 0, hh * _HBT + bt, 0, 0))
    return in_specs, out_spec


_OUT6 = jax.ShapeDtypeStruct((2, 6, _NCT, _NBT, 8, 128), jnp.float32)


@jax.jit
def _assemble_first(g, ma_l, pa_l):
    in_specs, out_spec = _mk_specs(0)
    return pl.pallas_call(
        _asm_body, grid=(_NSLOT, _HBT), in_specs=in_specs,
        out_specs=out_spec, out_shape=_OUT6,
    )(g, ma_l, pa_l)


@functools.partial(jax.jit, static_argnames=("part",))
def _assemble_next(g, ma_l, pa_l, prev, part):
    in_specs, out_spec = _mk_specs(part)
    in_specs.append(pl.BlockSpec(memory_space=pl.ANY))
    return pl.pallas_call(
        _asm_body, grid=(_NSLOT, _HBT), in_specs=in_specs,
        out_specs=out_spec, out_shape=_OUT6,
        input_output_aliases={3: 0},
    )(g, ma_l, pa_l, prev)


def kernel(fields, sides, species, moves, items, abilities, move_attributes,
           pokemon_attributes, species_table, item_table, ability_table,
           move_table):
    sp_i = jnp.transpose(species, (2, 1, 0)).reshape(12, _B)
    it_i = jnp.transpose(items, (2, 1, 0)).reshape(12, _B)
    ab_i = jnp.transpose(abilities, (2, 1, 0)).reshape(12, _B)
    mv_i = jnp.transpose(moves, (1, 2, 3, 0)).reshape(48, _B)
    idx_all = jnp.concatenate([sp_i, it_i, ab_i, mv_i], axis=0)
    it_tp = jnp.pad(item_table, ((0, 0), (0, 64)))
    ab_tp = jnp.pad(ability_table, ((0, 0), (0, 64)))
    ma_l = (move_attributes.transpose(1, 2, 3, 0, 4)
            .reshape(2, 6, 4, _NBT, 128, 8).transpose(0, 1, 2, 3, 5, 4))
    pa_l = (pokemon_attributes.transpose(1, 2, 3, 0)
            .reshape(2, 6, 6, 8, _NBT, 128).transpose(0, 1, 2, 4, 3, 5))
    o = None
    for part in range(_NP):
        idx_p = lax.slice_in_dim(idx_all, part * _HB, (part + 1) * _HB, axis=1)
        g = _gather(idx_p, species_table, move_table, it_tp, ab_tp)
        if part == 0:
            o = _assemble_first(g, ma_l, pa_l)
        else:
            o = _assemble_next(g, ma_l, pa_l, o, part=part)
    out = o.transpose(3, 5, 0, 1, 2, 4).reshape(_B, 2, 6, _ROW)
    return (fields, sides, out)
